# baseline (speedup 1.0000x reference)
import jax
import jax.numpy as jnp
from jax import lax
from jax.experimental import pallas as pl
from jax.experimental.pallas import tpu as pltpu
from jax.experimental.pallas import tpu_sc as plsc

_WINDOW = 128
_K = 8
_U = 16


def _repack_table(wt):
    n = wt.shape[1]
    grid = (n + 128 * _U - 1) // (128 * _U)

    def body(x_ref, o_ref):
        for u in range(_U):
            o_ref[pl.ds(128 * (u // 4), 128), pl.ds(32 * (u % 4), 32)] = (
                x_ref[:, pl.ds(128 * u, 128)].T)

    return pl.pallas_call(
        body,
        grid=(grid,),
        in_specs=[pl.BlockSpec((32, 128 * _U), lambda g: (0, g))],
        out_specs=pl.BlockSpec((128 * _U // 4, 128), lambda g: (g, 0)),
        out_shape=jax.ShapeDtypeStruct((grid * 128 * _U // 4, 128), wt.dtype),
    )(wt)


def _permute_indices(i):
    m = i >> 7
    r = i & 127
    g = m >> 4
    u = m & 15
    return 2048 * g + 512 * (u >> 2) + 4 * r + (u & 3)


def _gather_tiled(table, idx, t_len, b_len, dim):
    n_tw = t_len * b_len // (_K * _WINDOW)
    out_rows = t_len * dim * b_len // 128
    mesh = plsc.VectorSubcoreMesh(core_axis_name="core",
                                  subcore_axis_name="subcore")

    @pl.kernel(out_type=jax.ShapeDtypeStruct((out_rows, 128), jnp.float32),
               mesh=mesh,
               scratch_types=[pltpu.VMEM((_K * _WINDOW, dim), jnp.float32),
                              pltpu.SemaphoreType.DMA],
               compiler_params=pltpu.CompilerParams(use_tc_tiling_on_sc=False,
                                                    needs_layout_passes=False))
    def gather(w_hbm, i_hbm, o_hbm, y, sem):
        lane = lax.iota(jnp.int32, 16)

        def body(ids, i_vmem, z_ref, y):
            tw, c8 = ids

            @pl.when(c8 == 0)
            def _():
                copies = [
                    pltpu.async_copy(w_hbm.at[i_vmem.at[j]],
                                     y.at[pl.ds(j * _WINDOW, _WINDOW)], sem)
                    for j in range(_K)
                ]
                for c in copies:
                    c.wait()

            cols = [jnp.broadcast_to(8 * c8 + r, (16,)).astype(jnp.int32)
                    for r in range(8)]
            for q in range(64):
                bw, r = q >> 3, q & 7
                for s in range(8):
                    v = plsc.load_gather(
                        y, [128 * bw + 16 * s + lane, cols[r]])
                    z_ref[q, pl.ds(16 * s, 16)] = v

        pltpu.emit_pipeline(
            body,
            grid=(n_tw, 4),
            in_specs=[pl.BlockSpec((_K, _WINDOW),
                                   index_map=lambda tw, c8: (tw, 0))],
            out_specs=[pl.BlockSpec(
                (64, 128),
                index_map=lambda tw, c8: ((tw // 4) * 16 + c8 * 4 + tw % 4, 0))],
            core_axis_name=("core", "subcore"),
            dimension_semantics=(pltpu.PARALLEL, pltpu.ARBITRARY),
            _explicit_indices=True,
        )(i_hbm, o_hbm, scratches=[y])

    return gather(table, idx)


def kernel(input, weight):
    b_len, t_len = input.shape
    dim = weight.shape[1]

    w4 = _repack_table(weight.T)
    table = w4.reshape(w4.shape[0] * 4, dim)

    idx = _permute_indices(input).T.reshape(b_len * t_len // _WINDOW, _WINDOW)
    p2 = _gather_tiled(table, idx, t_len, b_len, dim)
    p5 = p2.reshape(t_len, dim // 8, b_len // 128, 8, 128)
    return p5.transpose(2, 4, 0, 1, 3).reshape(b_len, t_len, dim)


# speedup vs baseline: 1.4518x; 1.4518x over previous
import jax
import jax.numpy as jnp
from jax import lax
from jax.experimental import pallas as pl
from jax.experimental.pallas import tpu as pltpu
from jax.experimental.pallas import tpu_sc as plsc

_WINDOW = 128
_K = 8
_U = 16


def _repack_table(wt):
    n = wt.shape[1]
    grid = (n + 128 * _U - 1) // (128 * _U)

    def body(x_ref, o_ref):
        for u in range(_U):
            o_ref[pl.ds(128 * (u // 4), 128), pl.ds(32 * (u % 4), 32)] = (
                x_ref[:, pl.ds(128 * u, 128)].T)

    return pl.pallas_call(
        body,
        grid=(grid,),
        in_specs=[pl.BlockSpec((32, 128 * _U), lambda g: (0, g))],
        out_specs=pl.BlockSpec((128 * _U // 4, 128), lambda g: (g, 0)),
        out_shape=jax.ShapeDtypeStruct((grid * 128 * _U // 4, 128), wt.dtype),
    )(wt)


def _permute_indices(i):
    m = i >> 7
    r = i & 127
    g = m >> 4
    u = m & 15
    return 2048 * g + 512 * (u >> 2) + 4 * r + (u & 3)


def _gather_tiled(table, idx, t_len, b_len, dim):
    n_tw = t_len * b_len // (_K * _WINDOW)
    out_rows = t_len * dim * b_len // 128
    mesh = plsc.VectorSubcoreMesh(core_axis_name="core",
                                  subcore_axis_name="subcore")

    n_sub = 32
    tw_per = n_tw // n_sub

    @pl.kernel(out_type=jax.ShapeDtypeStruct((out_rows, 128), jnp.float32),
               mesh=mesh,
               scratch_types=[pltpu.VMEM((2 * _K * _WINDOW, dim), jnp.float32),
                              pltpu.VMEM((_K, _WINDOW), jnp.int32),
                              pltpu.SemaphoreType.DMA((2,)),
                              pltpu.SemaphoreType.DMA],
               compiler_params=pltpu.CompilerParams(use_tc_tiling_on_sc=False,
                                                    needs_layout_passes=False))
    def gather(w_hbm, i_hbm, o_hbm, y, idx0, sem, sem0):
        lane = lax.iota(jnp.int32, 16)
        wid = lax.axis_index("core") * 16 + lax.axis_index("subcore")
        tw0 = wid * tw_per
        nbuf = _K * _WINDOW

        pltpu.async_copy(i_hbm.at[pl.ds(tw0 * _K, _K)], idx0, sem0).wait()
        p0 = (tw0 & 1) * nbuf
        for j in range(_K):
            pltpu.async_copy(w_hbm.at[idx0.at[j]],
                             y.at[pl.ds(p0 + j * _WINDOW, _WINDOW)],
                             sem.at[tw0 & 1])

        def body(ids, i_vmem, z_ref, y, idx0):
            tw, c8 = ids
            base = (tw & 1) * nbuf

            @pl.when(c8 == 0)
            def _():
                nxt = nbuf - base
                for j in range(_K):
                    pltpu.async_copy(w_hbm.at[i_vmem.at[j]],
                                     y.at[pl.ds(nxt + j * _WINDOW, _WINDOW)],
                                     sem.at[(tw + 1) & 1])
                for j in range(_K):
                    pltpu.make_async_copy(
                        w_hbm.at[i_vmem.at[j]],
                        y.at[pl.ds(base + j * _WINDOW, _WINDOW)],
                        sem.at[tw & 1]).wait()

            @plsc.parallel_loop(0, 64, unroll=4)
            def _(q):
                bw = q >> 3
                r = q & 7
                col = jnp.broadcast_to(8 * c8 + r, (16,)).astype(jnp.int32)
                for s in range(8):
                    v = plsc.load_gather(
                        y, [base + 128 * bw + 16 * s + lane, col])
                    z_ref[q, pl.ds(16 * s, 16)] = v

        pltpu.emit_pipeline(
            body,
            grid=(n_tw, 4),
            in_specs=[pl.BlockSpec(
                (_K, _WINDOW),
                index_map=lambda tw, c8: (jnp.minimum(tw + 1, n_tw - 1), 0))],
            out_specs=[pl.BlockSpec(
                (64, 128),
                index_map=lambda tw, c8: ((tw // 4) * 16 + c8 * 4 + tw % 4, 0))],
            core_axis_name=("core", "subcore"),
            dimension_semantics=(pltpu.PARALLEL, pltpu.ARBITRARY),
            _explicit_indices=True,
        )(i_hbm, o_hbm, scratches=[y, idx0])

        p_last = ((tw0 + tw_per) & 1) * nbuf
        for j in range(_K):
            pltpu.make_async_copy(
                w_hbm.at[idx0.at[j]],
                y.at[pl.ds(p_last + j * _WINDOW, _WINDOW)],
                sem.at[(tw0 + tw_per) & 1]).wait()

    return gather(table, idx)


def kernel(input, weight):
    b_len, t_len = input.shape
    dim = weight.shape[1]

    w4 = _repack_table(weight.T)
    table = w4.reshape(w4.shape[0] * 4, dim)

    idx = _permute_indices(input).T.reshape(b_len * t_len // _WINDOW, _WINDOW)
    p2 = _gather_tiled(table, idx, t_len, b_len, dim)
    p5 = p2.reshape(t_len, dim // 8, b_len // 128, 8, 128)
    return p5.transpose(2, 4, 0, 1, 3).reshape(b_len, t_len, dim)
